# baseline (speedup 1.0000x reference)
import functools

import jax
import jax.numpy as jnp
from jax import lax
from jax.experimental import pallas as pl
from jax.experimental.pallas import tpu as pltpu
from jax.experimental.pallas import tpu_sc as plsc

NUM_CLASSES = 64
D = 128
N = 320000
NC, NS = 2, 16
NW = NC * NS
BLK = 128
NB = N // BLK
ITERS = (NB + NW - 1) // NW
SAMP = ((NB + 16 + 7) // 8) * 8
LPAD = BLK + 16
NBUF = 6


def _first_ge1(lref, c):
  def step(_, carry):
    lo, hi = carry
    mid = (lo + hi) // 2
    pred = lref[pl.ds(mid, 16)][0] < c
    return jnp.where(pred, mid + 1, lo), jnp.where(pred, hi, mid)

  _, hi = lax.fori_loop(0, 7, step, (jnp.int32(0), jnp.int32(BLK)))
  return hi


def _pos_of_class(samp_v, lab_blocks_hbm, lblk, c):
  def step(_, carry):
    lo, hi = carry
    mid = (lo + hi) // 2
    pred = samp_v[pl.ds(mid, 16)][0] < c
    return jnp.where(pred, mid + 1, lo), jnp.where(pred, hi, mid)

  lo, hi = lax.fori_loop(0, 12, step, (jnp.int32(0), jnp.int32(NB)))
  b = jnp.maximum(hi - 1, 0)
  pltpu.sync_copy(lab_blocks_hbm.at[b], lblk.at[pl.ds(0, BLK)])
  return (b * BLK + _first_ge1(lblk, c)).astype(jnp.float32)


def _sc_body(feat_hbm, lab_blocks_hbm, samp_hbm,
             sums_out, cnts_out, fblk2, lb_a, lb_b, lb_c, lb_d, lb_e,
             lb_f, lblk, samp_v, cvec, acc2, gsem):
  cid = lax.axis_index("c")
  sid = lax.axis_index("s")
  wid = sid * NC + cid
  lbufs = (lb_a, lb_b, lb_c, lb_d, lb_e, lb_f)

  def gather_start(bid, b):
    pltpu.async_copy(feat_hbm.at[bid], fblk2.at[b], gsem)
    pltpu.async_copy(lab_blocks_hbm.at[bid], lbufs[b].at[pl.ds(0, BLK)],
                     gsem)

  def gather_wait(b):
    pltpu.make_async_copy(feat_hbm.at[0], fblk2.at[b], gsem).wait()
    pltpu.make_async_copy(lab_blocks_hbm.at[0], lbufs[b].at[pl.ds(0, BLK)],
                          gsem).wait()

  for m in range(NBUF - 1):
    gather_start(m * NW + wid, m)

  pltpu.sync_copy(samp_hbm, samp_v)

  c0 = 2 * wid
  p0 = _pos_of_class(samp_v, lab_blocks_hbm, lblk, c0)
  p1 = _pos_of_class(samp_v, lab_blocks_hbm, lblk, c0 + 1)
  p2 = _pos_of_class(samp_v, lab_blocks_hbm, lblk, c0 + 2)
  lane = lax.iota(jnp.int32, 16)
  cnts = jnp.where(lane == 0, p1 - p0, jnp.where(lane == 1, p2 - p1, 0.0))
  cvec[...] = cnts
  pltpu.sync_copy(cvec, cnts_out.at[wid])

  zeros16 = jnp.zeros((16,), jnp.float32)

  @pl.loop(0, NUM_CLASSES)
  def _(i):
    for k in range(D // 16):
      acc2[i, pl.ds(k * 16, 16)] = zeros16

  @pl.loop(0, ITERS, step=NBUF)
  def _(j0):
    for b in range(NBUF):
      j = j0 + b
      bid = j * NW + wid

      @pl.when(bid < NB)
      def _():
        @pl.when(bid + (NBUF - 1) * NW < NB)
        def _():
          gather_start(bid + (NBUF - 1) * NW, (b + NBUF - 1) % NBUF)

        gather_wait(b)

        lb = lbufs[b]
        first = lb[pl.ds(0, 16)][0]
        last = lb[pl.ds(BLK - 1, 16)][0]
        zvs = tuple(jnp.zeros((16,), jnp.float32) for _ in range(D // 16))

        def flush(cls, vs):
          for k in range(D // 16):
            acc2[cls, pl.ds(k * 16, 16)] = (
                acc2[cls, pl.ds(k * 16, 16)] + vs[k])

        @pl.when(first == last)
        def _():
          def row8(i, vs):
            for u in range(8):
              r = i * 8 + u
              vs = tuple(
                  vs[k] + fblk2[b, r, pl.ds(k * 16, 16)]
                  for k in range(D // 16))
            return vs

          vs = lax.fori_loop(0, BLK // 8, row8, zvs)
          flush(first, vs)

        @pl.when(first != last)
        def _():
          def per_class(t, s):
            cls = first + t
            e = _first_ge1(lb, cls + 1)

            def per_row(r, vs):
              return tuple(
                  vs[k] + fblk2[b, r, pl.ds(k * 16, 16)]
                  for k in range(D // 16))

            vs = lax.fori_loop(s, e, per_row, zvs)
            flush(cls, vs)
            return e

          lax.fori_loop(0, last - first + 1, per_class, jnp.int32(0))

  pltpu.sync_copy(acc2, sums_out.at[wid])


_sc_segment_sums = functools.partial(
    pl.kernel,
    out_type=(
        jax.ShapeDtypeStruct((NW, NUM_CLASSES, D), jnp.float32),
        jax.ShapeDtypeStruct((NW, 16), jnp.float32),
    ),
    mesh=plsc.VectorSubcoreMesh(core_axis_name="c", subcore_axis_name="s",
                                num_cores=NC, num_subcores=NS),
    scratch_types=[
        pltpu.VMEM((NBUF, BLK, D), jnp.float32),
        pltpu.VMEM((LPAD,), jnp.int32),
        pltpu.VMEM((LPAD,), jnp.int32),
        pltpu.VMEM((LPAD,), jnp.int32),
        pltpu.VMEM((LPAD,), jnp.int32),
        pltpu.VMEM((LPAD,), jnp.int32),
        pltpu.VMEM((LPAD,), jnp.int32),
        pltpu.VMEM((LPAD,), jnp.int32),
        pltpu.VMEM((SAMP,), jnp.int32),
        pltpu.VMEM((16,), jnp.float32),
        pltpu.VMEM((NUM_CLASSES, D), jnp.float32),
        pltpu.SemaphoreType.DMA,
    ],
)(_sc_body)


def _combine_body(sums_ref, cnts_ref, out_ref):
  s = jnp.sum(sums_ref[...], axis=0)
  out_ref[...] = s / cnts_ref[...]


def kernel(support_features, support_labels):
  feat = support_features.reshape(NB, BLK, D)
  lab = support_labels.astype(jnp.int32).reshape(NB, BLK)
  samp = jnp.pad(lab[:, 0], (0, SAMP - NB))

  sums, cnts = _sc_segment_sums(feat, lab, samp)
  counts_col = cnts[:, :2].reshape(NUM_CLASSES, 1)

  return pl.pallas_call(
      _combine_body,
      out_shape=jax.ShapeDtypeStruct((NUM_CLASSES, D), jnp.float32),
  )(sums, counts_col)


# speedup vs baseline: 1.0243x; 1.0243x over previous
import functools

import jax
import jax.numpy as jnp
from jax import lax
from jax.experimental import pallas as pl
from jax.experimental.pallas import tpu as pltpu
from jax.experimental.pallas import tpu_sc as plsc

NUM_CLASSES = 64
D = 128
N = 320000
NC, NS = 2, 16
NW = NC * NS
BLK = 256
NB = N // BLK
ITERS = (NB + NW - 1) // NW
SAMP = ((NB + 16 + 7) // 8) * 8
LPAD = BLK + 16
NBUF = 3


def _first_ge1(lref, c):
  def step(_, carry):
    lo, hi = carry
    mid = (lo + hi) // 2
    pred = lref[pl.ds(mid, 16)][0] < c
    return jnp.where(pred, mid + 1, lo), jnp.where(pred, hi, mid)

  _, hi = lax.fori_loop(0, 8, step, (jnp.int32(0), jnp.int32(BLK)))
  return hi


def _pos_of_class(samp_v, lab_blocks_hbm, lblk, c):
  def step(_, carry):
    lo, hi = carry
    mid = (lo + hi) // 2
    pred = samp_v[pl.ds(mid, 16)][0] < c
    return jnp.where(pred, mid + 1, lo), jnp.where(pred, hi, mid)

  lo, hi = lax.fori_loop(0, 12, step, (jnp.int32(0), jnp.int32(NB)))
  b = jnp.maximum(hi - 1, 0)
  pltpu.sync_copy(lab_blocks_hbm.at[b], lblk.at[pl.ds(0, BLK)])
  return (b * BLK + _first_ge1(lblk, c)).astype(jnp.float32)


def _sc_body(feat_hbm, lab_blocks_hbm, samp_hbm,
             sums_out, cnts_out, fblk2, lb_a, lb_b, lb_c, lblk,
             samp_v, cvec, acc2, gsem):
  cid = lax.axis_index("c")
  sid = lax.axis_index("s")
  wid = sid * NC + cid
  lbufs = (lb_a, lb_b, lb_c)

  def gather_start(bid, b):
    pltpu.async_copy(feat_hbm.at[bid], fblk2.at[b], gsem)
    pltpu.async_copy(lab_blocks_hbm.at[bid], lbufs[b].at[pl.ds(0, BLK)],
                     gsem)

  def gather_wait(b):
    pltpu.make_async_copy(feat_hbm.at[0], fblk2.at[b], gsem).wait()
    pltpu.make_async_copy(lab_blocks_hbm.at[0], lbufs[b].at[pl.ds(0, BLK)],
                          gsem).wait()

  for m in range(NBUF - 1):
    gather_start(m * NW + wid, m)

  pltpu.sync_copy(samp_hbm, samp_v)

  c0 = 2 * wid
  p0 = _pos_of_class(samp_v, lab_blocks_hbm, lblk, c0)
  p1 = _pos_of_class(samp_v, lab_blocks_hbm, lblk, c0 + 1)
  p2 = _pos_of_class(samp_v, lab_blocks_hbm, lblk, c0 + 2)
  lane = lax.iota(jnp.int32, 16)
  cnts = jnp.where(lane == 0, p1 - p0, jnp.where(lane == 1, p2 - p1, 0.0))
  cvec[...] = cnts
  pltpu.sync_copy(cvec, cnts_out.at[wid])

  zeros16 = jnp.zeros((16,), jnp.float32)

  @pl.loop(0, NUM_CLASSES)
  def _(i):
    for k in range(D // 16):
      acc2[i, pl.ds(k * 16, 16)] = zeros16

  @pl.loop(0, ITERS, step=NBUF)
  def _(j0):
    for b in range(NBUF):
      j = j0 + b
      bid = j * NW + wid

      @pl.when(bid < NB)
      def _():
        @pl.when(bid + (NBUF - 1) * NW < NB)
        def _():
          gather_start(bid + (NBUF - 1) * NW, (b + NBUF - 1) % NBUF)

        gather_wait(b)

        lb = lbufs[b]
        first = lb[pl.ds(0, 16)][0]
        last = lb[pl.ds(BLK - 1, 16)][0]
        zvs = tuple(jnp.zeros((16,), jnp.float32) for _ in range(D // 16))

        def flush(cls, vs):
          for k in range(D // 16):
            acc2[cls, pl.ds(k * 16, 16)] = (
                acc2[cls, pl.ds(k * 16, 16)] + vs[k])

        @pl.when(first == last)
        def _():
          def row8(i, vs):
            for u in range(8):
              r = i * 8 + u
              vs = tuple(
                  vs[k] + fblk2[b, r, pl.ds(k * 16, 16)]
                  for k in range(D // 16))
            return vs

          vs = lax.fori_loop(0, BLK // 8, row8, zvs)
          flush(first, vs)

        @pl.when(first != last)
        def _():
          def per_class(t, s):
            cls = first + t
            e = _first_ge1(lb, cls + 1)

            def per_row(r, vs):
              return tuple(
                  vs[k] + fblk2[b, r, pl.ds(k * 16, 16)]
                  for k in range(D // 16))

            vs = lax.fori_loop(s, e, per_row, zvs)
            flush(cls, vs)
            return e

          lax.fori_loop(0, last - first + 1, per_class, jnp.int32(0))

  pltpu.sync_copy(acc2, sums_out.at[wid])


_sc_segment_sums = functools.partial(
    pl.kernel,
    out_type=(
        jax.ShapeDtypeStruct((NW, NUM_CLASSES, D), jnp.float32),
        jax.ShapeDtypeStruct((NW, 16), jnp.float32),
    ),
    mesh=plsc.VectorSubcoreMesh(core_axis_name="c", subcore_axis_name="s",
                                num_cores=NC, num_subcores=NS),
    scratch_types=[
        pltpu.VMEM((NBUF, BLK, D), jnp.float32),
        pltpu.VMEM((LPAD,), jnp.int32),
        pltpu.VMEM((LPAD,), jnp.int32),
        pltpu.VMEM((LPAD,), jnp.int32),
        pltpu.VMEM((LPAD,), jnp.int32),
        pltpu.VMEM((SAMP,), jnp.int32),
        pltpu.VMEM((16,), jnp.float32),
        pltpu.VMEM((NUM_CLASSES, D), jnp.float32),
        pltpu.SemaphoreType.DMA,
    ],
)(_sc_body)


def _combine_body(sums_ref, cnts_ref, out_ref):
  s = jnp.sum(sums_ref[...], axis=0)
  out_ref[...] = s / cnts_ref[...]


def kernel(support_features, support_labels):
  feat = support_features.reshape(NB, BLK, D)
  lab = support_labels.astype(jnp.int32).reshape(NB, BLK)
  samp = jnp.pad(lab[:, 0], (0, SAMP - NB))

  sums, cnts = _sc_segment_sums(feat, lab, samp)
  counts_col = cnts[:, :2].reshape(NUM_CLASSES, 1)

  return pl.pallas_call(
      _combine_body,
      out_shape=jax.ShapeDtypeStruct((NUM_CLASSES, D), jnp.float32),
  )(sums, counts_col)
